# baseline (speedup 1.0000x reference)
import functools

import jax
import jax.numpy as jnp
from jax import lax
from jax.experimental import pallas as pl
from jax.experimental.pallas import tpu as pltpu
from jax.experimental.pallas import tpu_sc as plsc

V = 100000
VP = 102400
H = 64
B = 4096
L = 200
CP = 16
NC, NS = 2, 16
NW = NC * NS
BPW = B // NW
C0, C1 = 104, 96
BLKV = 12800


def _tc_table_body(emb_ref, w1_ref, w2p_ref, b1_ref, b2p_ref, t2T_ref, c_ref):
    mT = lax.dot_general(w2p_ref[...], w1_ref[...],
                         (((1,), (0,)), ((), ())),
                         preferred_element_type=jnp.float32)
    t2T_ref[...] = lax.dot_general(mT, emb_ref[...],
                                   (((1,), (1,)), ((), ())),
                                   preferred_element_type=jnp.float32)

    @pl.when(pl.program_id(0) == 0)
    def _():
        c_ref[...] = lax.dot_general(b1_ref[...], w2p_ref[...],
                                     (((1,), (1,)), ((), ())),
                                     preferred_element_type=jnp.float32) + b2p_ref[...]


_tc_table = pl.pallas_call(
    _tc_table_body,
    grid=(VP // BLKV,),
    in_specs=[
        pl.BlockSpec((BLKV, H), lambda i: (i, 0)),
        pl.BlockSpec((H, H), lambda i: (0, 0)),
        pl.BlockSpec((CP, H), lambda i: (0, 0)),
        pl.BlockSpec((1, H), lambda i: (0, 0)),
        pl.BlockSpec((1, CP), lambda i: (0, 0)),
    ],
    out_specs=[
        pl.BlockSpec((CP, BLKV), lambda i: (0, i)),
        pl.BlockSpec((1, CP), lambda i: (0, 0)),
    ],
    out_shape=[
        jax.ShapeDtypeStruct((CP, VP), jnp.float32),
        jax.ShapeDtypeStruct((1, CP), jnp.float32),
    ],
)


def _reduce_rows(buf):
    a0 = buf[0] + buf[4]
    a1 = buf[1] + buf[5]
    a2 = buf[2] + buf[6]
    a3 = buf[3] + buf[7]
    for j in range(8, L, 8):
        a0 = a0 + buf[j] + buf[j + 4]
        a1 = a1 + buf[j + 1] + buf[j + 5]
        a2 = a2 + buf[j + 2] + buf[j + 6]
        a3 = a3 + buf[j + 3] + buf[j + 7]
    return (a0 + a1) + (a2 + a3)


@functools.partial(
    pl.kernel,
    out_type=jax.ShapeDtypeStruct((B, CP), jnp.float32),
    mesh=plsc.VectorSubcoreMesh(core_axis_name="c", subcore_axis_name="s",
                                num_cores=NC, num_subcores=NS),
    compiler_params=pltpu.CompilerParams(use_tc_tiling_on_sc=False),
    scratch_types=[
        pltpu.VMEM((BPW * L,), jnp.int32),
        pltpu.VMEM((L, CP), jnp.float32),
        pltpu.VMEM((L, CP), jnp.float32),
        pltpu.VMEM((L, CP), jnp.float32),
        pltpu.VMEM((L, CP), jnp.float32),
        pltpu.VMEM((BPW, CP), jnp.float32),
        pltpu.VMEM((CP,), jnp.float32),
        pltpu.SemaphoreType.DMA,
        pltpu.SemaphoreType.DMA,
        pltpu.SemaphoreType.DMA,
        pltpu.SemaphoreType.DMA,
    ],
)
def _sc_pool(xf_hbm, t2_hbm, c_hbm, out_hbm,
             idx_v, buf0, buf1, buf2, buf3, out_v, c_v,
             sem0, sem1, sem2, sem3):
    wid = lax.axis_index("s") * NC + lax.axis_index("c")
    base = wid * BPW
    bufs = (buf0, buf1, buf2, buf3)
    sems = (sem0, sem1, sem2, sem3)
    ND = 4

    pltpu.sync_copy(xf_hbm.at[pl.ds(base * L, BPW * L)], idx_v)
    pltpu.sync_copy(c_hbm, c_v)
    cvec = c_v[...]
    scale = jnp.float32(1.0 / L)

    def fire(row, buf, sem):
        off = row * L
        pltpu.async_copy(t2_hbm.at[idx_v.at[pl.ds(off, C0)]],
                         buf.at[pl.ds(0, C0)], sem)
        pltpu.async_copy(t2_hbm.at[idx_v.at[pl.ds(off + C0, C1)]],
                         buf.at[pl.ds(C0, C1)], sem)

    def drain(buf, sem):
        pltpu.make_async_copy(t2_hbm.at[pl.ds(0, L)], buf, sem).wait()

    for t in range(ND):
        fire(t, bufs[t], sems[t])

    def quad_body(q, _):
        r0 = ND * q
        for t in range(ND):
            r = r0 + t
            drain(bufs[t], sems[t])
            out_v[r, :] = _reduce_rows(bufs[t]) * scale + cvec
            fire(jnp.minimum(r + ND, BPW - 1), bufs[t], sems[t])
        return 0

    lax.fori_loop(0, BPW // ND, quad_body, 0)
    for t in range(ND):
        drain(bufs[t], sems[t])

    pltpu.sync_copy(out_v, out_hbm.at[pl.ds(base, BPW)])


def kernel(x, emb, W1, b1, W2, b2):
    w2p = jnp.zeros((CP, H), jnp.float32).at[: W2.shape[0]].set(W2)
    b2p = jnp.zeros((1, CP), jnp.float32).at[0, : b2.shape[0]].set(b2)
    t2T, c = _tc_table(emb, W1, w2p, b1.reshape(1, H), b2p)
    xf = x.reshape(-1).astype(jnp.int32)
    out16 = _sc_pool(xf, t2T.T, c.reshape(CP))
    return out16[:, : W2.shape[0]]


# speedup vs baseline: 1.3392x; 1.3392x over previous
import functools

import jax
import jax.numpy as jnp
from jax import lax
from jax.experimental import pallas as pl
from jax.experimental.pallas import tpu as pltpu
from jax.experimental.pallas import tpu_sc as plsc

V = 100000
VP = 102400
H = 64
B = 4096
L = 200
CP = 16
NC, NS = 2, 16
NW = NC * NS
BPW = B // NW
C0, C1 = 104, 96
G8 = 8
VG = VP // G8
VG_REAL = V * H // (G8 * H)
BLKG = 1600


def _tc_table_body(emb8_ref, w1_ref, w2p_ref, b1_ref, b2p_ref, o_ref, c_ref):
    m = lax.dot_general(w1_ref[...], w2p_ref[...],
                        (((0,), (1,)), ((), ())),
                        preferred_element_type=jnp.float32)
    mt = jnp.tile(m, (G8, G8))
    ki = lax.broadcasted_iota(jnp.int32, (G8 * H, G8 * CP), 0)
    ji = lax.broadcasted_iota(jnp.int32, (G8 * H, G8 * CP), 1)
    m8 = jnp.where((ki // H) == (ji // CP), mt, jnp.float32(0.0))
    o_ref[...] = lax.dot_general(emb8_ref[...], m8,
                                 (((1,), (0,)), ((), ())),
                                 preferred_element_type=jnp.float32)

    @pl.when(pl.program_id(0) == 0)
    def _():
        c_ref[...] = lax.dot_general(b1_ref[...], w2p_ref[...],
                                     (((1,), (1,)), ((), ())),
                                     preferred_element_type=jnp.float32) + b2p_ref[...]


_tc_table = pl.pallas_call(
    _tc_table_body,
    grid=(VG // BLKG,),
    in_specs=[
        pl.BlockSpec((BLKG, G8 * H), lambda i: (i, 0)),
        pl.BlockSpec((H, H), lambda i: (0, 0)),
        pl.BlockSpec((CP, H), lambda i: (0, 0)),
        pl.BlockSpec((1, H), lambda i: (0, 0)),
        pl.BlockSpec((1, CP), lambda i: (0, 0)),
    ],
    out_specs=[
        pl.BlockSpec((BLKG, G8 * CP), lambda i: (i, 0)),
        pl.BlockSpec((1, CP), lambda i: (0, 0)),
    ],
    out_shape=[
        jax.ShapeDtypeStruct((VG, G8 * CP), jnp.float32),
        jax.ShapeDtypeStruct((1, CP), jnp.float32),
    ],
)


def _reduce_rows(buf):
    a0 = buf[0] + buf[4]
    a1 = buf[1] + buf[5]
    a2 = buf[2] + buf[6]
    a3 = buf[3] + buf[7]
    for j in range(8, L, 8):
        a0 = a0 + buf[j] + buf[j + 4]
        a1 = a1 + buf[j + 1] + buf[j + 5]
        a2 = a2 + buf[j + 2] + buf[j + 6]
        a3 = a3 + buf[j + 3] + buf[j + 7]
    return (a0 + a1) + (a2 + a3)


@functools.partial(
    pl.kernel,
    out_type=jax.ShapeDtypeStruct((B, CP), jnp.float32),
    mesh=plsc.VectorSubcoreMesh(core_axis_name="c", subcore_axis_name="s",
                                num_cores=NC, num_subcores=NS),
    compiler_params=pltpu.CompilerParams(use_tc_tiling_on_sc=False),
    scratch_types=[
        pltpu.VMEM((BPW * L,), jnp.int32),
        pltpu.VMEM((L, CP), jnp.float32),
        pltpu.VMEM((L, CP), jnp.float32),
        pltpu.VMEM((L, CP), jnp.float32),
        pltpu.VMEM((L, CP), jnp.float32),
        pltpu.VMEM((BPW, CP), jnp.float32),
        pltpu.VMEM((CP,), jnp.float32),
        pltpu.SemaphoreType.DMA,
        pltpu.SemaphoreType.DMA,
        pltpu.SemaphoreType.DMA,
        pltpu.SemaphoreType.DMA,
    ],
)
def _sc_pool(xf_hbm, t2_hbm, c_hbm, out_hbm,
             idx_v, buf0, buf1, buf2, buf3, out_v, c_v,
             sem0, sem1, sem2, sem3):
    wid = lax.axis_index("s") * NC + lax.axis_index("c")
    base = wid * BPW
    bufs = (buf0, buf1, buf2, buf3)
    sems = (sem0, sem1, sem2, sem3)
    ND = 4

    pltpu.sync_copy(xf_hbm.at[pl.ds(base * L, BPW * L)], idx_v)
    pltpu.sync_copy(c_hbm, c_v)
    cvec = c_v[...]
    scale = jnp.float32(1.0 / L)

    def fire(row, buf, sem):
        off = row * L
        pltpu.async_copy(t2_hbm.at[idx_v.at[pl.ds(off, C0)]],
                         buf.at[pl.ds(0, C0)], sem)
        pltpu.async_copy(t2_hbm.at[idx_v.at[pl.ds(off + C0, C1)]],
                         buf.at[pl.ds(C0, C1)], sem)

    def drain(buf, sem):
        pltpu.make_async_copy(t2_hbm.at[pl.ds(0, L)], buf, sem).wait()

    for t in range(ND):
        fire(t, bufs[t], sems[t])

    def quad_body(q, _):
        r0 = ND * q
        for t in range(ND):
            r = r0 + t
            drain(bufs[t], sems[t])
            out_v[r, :] = _reduce_rows(bufs[t]) * scale + cvec
            fire(jnp.minimum(r + ND, BPW - 1), bufs[t], sems[t])
        return 0

    lax.fori_loop(0, BPW // ND, quad_body, 0)
    for t in range(ND):
        drain(bufs[t], sems[t])

    pltpu.sync_copy(out_v, out_hbm.at[pl.ds(base, BPW)])


def kernel(x, emb, W1, b1, W2, b2):
    w2p = jnp.zeros((CP, H), jnp.float32).at[: W2.shape[0]].set(W2)
    b2p = jnp.zeros((1, CP), jnp.float32).at[0, : b2.shape[0]].set(b2)
    emb8 = emb.reshape(VG_REAL, G8 * H)
    o, c = _tc_table(emb8, W1, w2p, b1.reshape(1, H), b2p)
    xf = x.reshape(-1).astype(jnp.int32)
    out16 = _sc_pool(xf, o.reshape(VP, CP), c.reshape(CP))
    return out16[:, : W2.shape[0]]


# speedup vs baseline: 1.4039x; 1.0483x over previous
import functools

import jax
import jax.numpy as jnp
from jax import lax
from jax.experimental import pallas as pl
from jax.experimental.pallas import tpu as pltpu
from jax.experimental.pallas import tpu_sc as plsc

V = 100000
VP = 102400
H = 64
B = 4096
L = 200
CP = 16
NC, NS = 2, 16
NW = NC * NS
BPW = B // NW
C0, C1 = 104, 96
G8 = 8
VG = VP // G8
BLKV = 12800


def _tc_table_body(emb_ref, w1_ref, w2p_ref, b1_ref, b2p_ref, o_ref, c_ref):
    m = lax.dot_general(w1_ref[...], w2p_ref[...],
                        (((0,), (1,)), ((), ())),
                        preferred_element_type=jnp.float32)
    m8 = jnp.tile(m, (1, G8))
    p8 = lax.dot_general(emb_ref[...], m8,
                         (((1,), (0,)), ((), ())),
                         preferred_element_type=jnp.float32)
    p83 = p8.reshape(BLKV // G8, G8, G8 * CP)
    si = lax.broadcasted_iota(jnp.int32, (G8, G8 * CP), 0)
    ji = lax.broadcasted_iota(jnp.int32, (G8, G8 * CP), 1)
    sel = jnp.where((ji // CP) == si, jnp.float32(1.0), jnp.float32(0.0))
    o_ref[...] = jnp.sum(p83 * sel[None, :, :], axis=1)

    @pl.when(pl.program_id(0) == 0)
    def _():
        c_ref[...] = lax.dot_general(b1_ref[...], w2p_ref[...],
                                     (((1,), (1,)), ((), ())),
                                     preferred_element_type=jnp.float32) + b2p_ref[...]


_tc_table = pl.pallas_call(
    _tc_table_body,
    grid=(VP // BLKV,),
    in_specs=[
        pl.BlockSpec((BLKV, H), lambda i: (i, 0)),
        pl.BlockSpec((H, H), lambda i: (0, 0)),
        pl.BlockSpec((CP, H), lambda i: (0, 0)),
        pl.BlockSpec((1, H), lambda i: (0, 0)),
        pl.BlockSpec((1, CP), lambda i: (0, 0)),
    ],
    out_specs=[
        pl.BlockSpec((BLKV // G8, G8 * CP), lambda i: (i, 0)),
        pl.BlockSpec((1, CP), lambda i: (0, 0)),
    ],
    out_shape=[
        jax.ShapeDtypeStruct((VG, G8 * CP), jnp.float32),
        jax.ShapeDtypeStruct((1, CP), jnp.float32),
    ],
)


def _reduce_rows(buf):
    a0 = buf[0] + buf[4]
    a1 = buf[1] + buf[5]
    a2 = buf[2] + buf[6]
    a3 = buf[3] + buf[7]
    for j in range(8, L, 8):
        a0 = a0 + buf[j] + buf[j + 4]
        a1 = a1 + buf[j + 1] + buf[j + 5]
        a2 = a2 + buf[j + 2] + buf[j + 6]
        a3 = a3 + buf[j + 3] + buf[j + 7]
    return (a0 + a1) + (a2 + a3)


@functools.partial(
    pl.kernel,
    out_type=jax.ShapeDtypeStruct((B, CP), jnp.float32),
    mesh=plsc.VectorSubcoreMesh(core_axis_name="c", subcore_axis_name="s",
                                num_cores=NC, num_subcores=NS),
    compiler_params=pltpu.CompilerParams(use_tc_tiling_on_sc=False),
    scratch_types=[
        pltpu.VMEM((BPW * L,), jnp.int32),
        pltpu.VMEM((L, CP), jnp.float32),
        pltpu.VMEM((L, CP), jnp.float32),
        pltpu.VMEM((L, CP), jnp.float32),
        pltpu.VMEM((L, CP), jnp.float32),
        pltpu.VMEM((BPW, CP), jnp.float32),
        pltpu.VMEM((CP,), jnp.float32),
        pltpu.SemaphoreType.DMA,
        pltpu.SemaphoreType.DMA,
        pltpu.SemaphoreType.DMA,
        pltpu.SemaphoreType.DMA,
    ],
)
def _sc_pool(xf_hbm, t2_hbm, c_hbm, out_hbm,
             idx_v, buf0, buf1, buf2, buf3, out_v, c_v,
             sem0, sem1, sem2, sem3):
    wid = lax.axis_index("s") * NC + lax.axis_index("c")
    base = wid * BPW
    bufs = (buf0, buf1, buf2, buf3)
    sems = (sem0, sem1, sem2, sem3)
    ND = 4

    pltpu.sync_copy(xf_hbm.at[pl.ds(base * L, BPW * L)], idx_v)
    pltpu.sync_copy(c_hbm, c_v)
    cvec = c_v[...]
    scale = jnp.float32(1.0 / L)

    def fire(row, buf, sem):
        off = row * L
        pltpu.async_copy(t2_hbm.at[idx_v.at[pl.ds(off, C0)]],
                         buf.at[pl.ds(0, C0)], sem)
        pltpu.async_copy(t2_hbm.at[idx_v.at[pl.ds(off + C0, C1)]],
                         buf.at[pl.ds(C0, C1)], sem)

    def drain(buf, sem):
        pltpu.make_async_copy(t2_hbm.at[pl.ds(0, L)], buf, sem).wait()

    for t in range(ND):
        fire(t, bufs[t], sems[t])

    def quad_body(q, _):
        r0 = ND * q
        for t in range(ND):
            r = r0 + t
            drain(bufs[t], sems[t])
            out_v[r, :] = _reduce_rows(bufs[t]) * scale + cvec
            fire(jnp.minimum(r + ND, BPW - 1), bufs[t], sems[t])
        return 0

    lax.fori_loop(0, BPW // ND, quad_body, 0)
    for t in range(ND):
        drain(bufs[t], sems[t])

    pltpu.sync_copy(out_v, out_hbm.at[pl.ds(base, BPW)])


def kernel(x, emb, W1, b1, W2, b2):
    w2p = jnp.zeros((CP, H), jnp.float32).at[: W2.shape[0]].set(W2)
    b2p = jnp.zeros((1, CP), jnp.float32).at[0, : b2.shape[0]].set(b2)
    o, c = _tc_table(emb, W1, w2p, b1.reshape(1, H), b2p)
    xf = x.reshape(-1).astype(jnp.int32)
    out16 = _sc_pool(xf, o.reshape(VP, CP), c.reshape(CP))
    return out16[:, : W2.shape[0]]


# speedup vs baseline: 1.9653x; 1.3999x over previous
import functools

import jax
import jax.numpy as jnp
from jax import lax
from jax.experimental import pallas as pl
from jax.experimental.pallas import tpu as pltpu
from jax.experimental.pallas import tpu_sc as plsc

V = 100000
VP = 102400
H = 64
B = 4096
L = 200
CP = 16
NC, NS = 2, 16
NW = NC * NS
BPW = B // NW
C0, C1 = 104, 96
G8 = 8
VG = VP // G8
BLKV = 12800


def _tc_table_body(embT_ref, w1_ref, w2p_ref, b1_ref, b2p_ref, o_ref, c_ref):
    mT = lax.dot_general(w2p_ref[...], w1_ref[...],
                         (((1,), (0,)), ((), ())),
                         preferred_element_type=jnp.float32)
    pT = lax.dot_general(mT, embT_ref[...],
                         (((1,), (0,)), ((), ())),
                         preferred_element_type=jnp.float32)
    ci = lax.broadcasted_iota(jnp.int32, (CP, G8 * CP), 0)
    ei = lax.broadcasted_iota(jnp.int32, (CP, G8 * CP), 1)
    eyet = jnp.where((ei % CP) == ci, jnp.float32(1.0), jnp.float32(0.0))
    p8 = lax.dot_general(pT, eyet,
                         (((0,), (0,)), ((), ())),
                         preferred_element_type=jnp.float32)
    p83 = p8.reshape(BLKV // G8, G8, G8 * CP)
    si = lax.broadcasted_iota(jnp.int32, (G8, G8 * CP), 0)
    ji = lax.broadcasted_iota(jnp.int32, (G8, G8 * CP), 1)
    sel = jnp.where((ji // CP) == si, jnp.float32(1.0), jnp.float32(0.0))
    o_ref[...] = jnp.sum(p83 * sel[None, :, :], axis=1)

    @pl.when(pl.program_id(0) == 0)
    def _():
        c_ref[...] = lax.dot_general(b1_ref[...], w2p_ref[...],
                                     (((1,), (1,)), ((), ())),
                                     preferred_element_type=jnp.float32) + b2p_ref[...]


_tc_table = pl.pallas_call(
    _tc_table_body,
    grid=(VP // BLKV,),
    in_specs=[
        pl.BlockSpec((H, BLKV), lambda i: (0, i)),
        pl.BlockSpec((H, H), lambda i: (0, 0)),
        pl.BlockSpec((CP, H), lambda i: (0, 0)),
        pl.BlockSpec((1, H), lambda i: (0, 0)),
        pl.BlockSpec((1, CP), lambda i: (0, 0)),
    ],
    out_specs=[
        pl.BlockSpec((BLKV // G8, G8 * CP), lambda i: (i, 0)),
        pl.BlockSpec((1, CP), lambda i: (0, 0)),
    ],
    out_shape=[
        jax.ShapeDtypeStruct((VG, G8 * CP), jnp.float32),
        jax.ShapeDtypeStruct((1, CP), jnp.float32),
    ],
)


def _reduce_rows(buf):
    a0 = buf[0] + buf[4]
    a1 = buf[1] + buf[5]
    a2 = buf[2] + buf[6]
    a3 = buf[3] + buf[7]
    for j in range(8, L, 8):
        a0 = a0 + buf[j] + buf[j + 4]
        a1 = a1 + buf[j + 1] + buf[j + 5]
        a2 = a2 + buf[j + 2] + buf[j + 6]
        a3 = a3 + buf[j + 3] + buf[j + 7]
    return (a0 + a1) + (a2 + a3)


@functools.partial(
    pl.kernel,
    out_type=jax.ShapeDtypeStruct((B, CP), jnp.float32),
    mesh=plsc.VectorSubcoreMesh(core_axis_name="c", subcore_axis_name="s",
                                num_cores=NC, num_subcores=NS),
    compiler_params=pltpu.CompilerParams(use_tc_tiling_on_sc=False),
    scratch_types=[
        pltpu.VMEM((BPW * L,), jnp.int32),
        pltpu.VMEM((L, CP), jnp.float32),
        pltpu.VMEM((L, CP), jnp.float32),
        pltpu.VMEM((L, CP), jnp.float32),
        pltpu.VMEM((L, CP), jnp.float32),
        pltpu.VMEM((BPW, CP), jnp.float32),
        pltpu.VMEM((CP,), jnp.float32),
        pltpu.SemaphoreType.DMA,
        pltpu.SemaphoreType.DMA,
        pltpu.SemaphoreType.DMA,
        pltpu.SemaphoreType.DMA,
    ],
)
def _sc_pool(xf_hbm, t2_hbm, c_hbm, out_hbm,
             idx_v, buf0, buf1, buf2, buf3, out_v, c_v,
             sem0, sem1, sem2, sem3):
    wid = lax.axis_index("s") * NC + lax.axis_index("c")
    base = wid * BPW
    bufs = (buf0, buf1, buf2, buf3)
    sems = (sem0, sem1, sem2, sem3)
    ND = 4

    pltpu.sync_copy(xf_hbm.at[pl.ds(base * L, BPW * L)], idx_v)
    pltpu.sync_copy(c_hbm, c_v)
    cvec = c_v[...]
    scale = jnp.float32(1.0 / L)

    def fire(row, buf, sem):
        off = row * L
        pltpu.async_copy(t2_hbm.at[idx_v.at[pl.ds(off, C0)]],
                         buf.at[pl.ds(0, C0)], sem)
        pltpu.async_copy(t2_hbm.at[idx_v.at[pl.ds(off + C0, C1)]],
                         buf.at[pl.ds(C0, C1)], sem)

    def drain(buf, sem):
        pltpu.make_async_copy(t2_hbm.at[pl.ds(0, L)], buf, sem).wait()

    for t in range(ND):
        fire(t, bufs[t], sems[t])

    def quad_body(q, _):
        r0 = ND * q
        for t in range(ND):
            r = r0 + t
            drain(bufs[t], sems[t])
            out_v[r, :] = _reduce_rows(bufs[t]) * scale + cvec
            fire(jnp.minimum(r + ND, BPW - 1), bufs[t], sems[t])
        return 0

    lax.fori_loop(0, BPW // ND, quad_body, 0)
    for t in range(ND):
        drain(bufs[t], sems[t])

    pltpu.sync_copy(out_v, out_hbm.at[pl.ds(base, BPW)])


def kernel(x, emb, W1, b1, W2, b2):
    w2p = jnp.zeros((CP, H), jnp.float32).at[: W2.shape[0]].set(W2)
    b2p = jnp.zeros((1, CP), jnp.float32).at[0, : b2.shape[0]].set(b2)
    o, c = _tc_table(emb.T, W1, w2p, b1.reshape(1, H), b2p)
    xf = x.reshape(-1).astype(jnp.int32)
    out16 = _sc_pool(xf, o.reshape(VP, CP), c.reshape(CP))
    return out16[:, : W2.shape[0]]
